# baseline (speedup 1.0000x reference)
import functools

import jax
import jax.numpy as jnp
from jax import lax
from jax.experimental import pallas as pl
from jax.experimental.pallas import tpu as pltpu
from jax.experimental.pallas import tpu_sc as plsc

_N = 10000
_E = 320000
_IN = 128
_AUG = 6
_DIN = _IN - _AUG
_HEADS = 8
_HID = 8
_OUT = 16

_D1 = 128
_F1 = 64
_D2 = 32
_F2 = 16

_NC = 2
_NS = 16
_NW = _NC * _NS
_EPT = _E // _NW
_B = 40
_CPT = _EPT // _B
_PAIRS = _CPT // 2
_RPT = _N // _NS


def _tc1_body(x_ref, mw1_ref, mb1_ref, mw2_ref, mb2_ref, w1_ref, as_ref,
              ad_ref, tab_ref, adst_ref):
    x = x_ref[...]
    orig = x[:, :_DIN]
    app = x[:, _DIN:]
    mean = jnp.mean(app, axis=0, keepdims=True)
    cent = app - mean
    var = jnp.sum(cent * cent, axis=0, keepdims=True) / (_N - 1)
    z = cent / jnp.sqrt(var)
    hm = jnp.maximum(
        jnp.dot(z, mw1_ref[...], preferred_element_type=jnp.float32)
        + mb1_ref[...], 0.0)
    s = jnp.dot(hm, mw2_ref[...], preferred_element_type=jnp.float32) + mb2_ref[...]
    scale = 1.0 / (1.0 + jnp.exp(-s))
    h = orig * (1.0 + scale)
    h1 = jnp.dot(h, w1_ref[...], preferred_element_type=jnp.float32)
    a_srcx = jnp.dot(h1, as_ref[...], preferred_element_type=jnp.float32)
    a_dstx = jnp.dot(h1, ad_ref[...], preferred_element_type=jnp.float32)
    tab_ref[...] = jnp.concatenate([h1, a_srcx], axis=1)
    adst_ref[...] = a_dstx


def _tc2_body(p_ref, b1_ref, w2_ref, as2_ref, ad2_ref, tab2_ref, adst2_ref):
    p = p_ref[0] + p_ref[1]
    numer = p[:, :_F1]
    dexp = p[:, _F1:]
    h1o = jnp.maximum(numer / (dexp + 1e-16) + b1_ref[...], 0.0)
    h2 = jnp.dot(h1o, w2_ref[...], preferred_element_type=jnp.float32)
    a2sx = jnp.dot(h2, as2_ref[...], preferred_element_type=jnp.float32)
    a2dx = jnp.dot(h2, ad2_ref[...], preferred_element_type=jnp.float32)
    tab2_ref[...] = jnp.concatenate([h2, a2sx], axis=1)
    adst2_ref[...] = a2dx


def _tc3_body(p_ref, b2_ref, out_ref):
    p = p_ref[0] + p_ref[1]
    numer = p[:, :_F2]
    den = p[:, _F2:_F2 + 1]
    o = numer / (den + 1e-16) + b2_ref[...]
    m = jnp.max(o, axis=1, keepdims=True)
    lse = jnp.log(jnp.sum(jnp.exp(o - m), axis=1, keepdims=True)) + m
    out_ref[...] = o - lse


_NB1 = 5
_IDXN = 10


def _sc_edge_body_l1(table_hbm, adst_hbm, src2_hbm, dst2_hbm, zeros_hbm,
                     out_hbm, *scr, D, F):
    sidxr, didxr = scr[0], scr[1]
    rows_ = scr[2:2 + _NB1]
    arows_ = scr[2 + _NB1:2 + 2 * _NB1]
    acc = scr[2 + 2 * _NB1]
    sg_ = scr[3 + 2 * _NB1:3 + 3 * _NB1]
    ss_ = scr[3 + 3 * _NB1:3 + 4 * _NB1]
    si_ = scr[3 + 4 * _NB1:3 + 4 * _NB1 + _IDXN]
    cid = lax.axis_index("c")
    sid = lax.axis_index("s")
    wid = sid * _NC + cid
    nblk = F // 16

    row0 = pl.multiple_of(sid * _RPT, _RPT)
    pltpu.sync_copy(zeros_hbm, acc.at[pl.ds(row0, _RPT)])
    plsc.subcore_barrier()

    crow = wid * _CPT
    for s in range(_IDXN):
        pltpu.async_copy(src2_hbm.at[crow + s], sidxr.at[s], si_[s])
        pltpu.async_copy(dst2_hbm.at[crow + s], didxr.at[s], si_[s])
    for b in range(3):
        pltpu.make_async_copy(src2_hbm.at[crow + b], sidxr.at[b], si_[b]).wait()
        pltpu.make_async_copy(dst2_hbm.at[crow + b], didxr.at[b], si_[b]).wait()
        pltpu.async_copy(table_hbm.at[sidxr.at[b]], rows_[b], sg_[b])
        pltpu.async_copy(adst_hbm.at[didxr.at[b]], arows_[b], sg_[b])

    def round_body(ri, carry):
        for u in range(_IDXN):
            ci = ri * _IDXN + u
            b = u % _NB1
            rws, ars = rows_[b], arows_[b]
            pltpu.make_async_copy(table_hbm.at[sidxr.at[u]], rws, sg_[b]).wait()
            pltpu.make_async_copy(adst_hbm.at[didxr.at[u]], ars, sg_[b]).wait()

            for e in range(_B):
                for k in range(nblk):
                    a = rws[e, pl.ds(F + k * 16, 16)] + ars[e, pl.ds(k * 16, 16)]
                    a = jnp.maximum(a, 0.2 * a)
                    wv = jnp.exp(a)
                    rws[e, pl.ds(F + k * 16, 16)] = wv
                    rws[e, pl.ds(k * 16, 16)] = rws[e, pl.ds(k * 16, 16)] * wv
            pltpu.async_copy(rws, acc.at[didxr.at[u]], ss_[b], add=True)

            b2 = (u + 3) % _NB1
            u2 = (u + 3) % _IDXN
            u8 = (u + 8) % _IDXN

            @pl.when(ci >= 2)
            def _wait_scatter():
                pltpu.make_async_copy(rows_[b2], acc.at[didxr.at[u2]],
                                      ss_[b2]).wait()

            @pl.when((ci >= 2) & (ci + 8 < _CPT))
            def _refetch_idx():
                pltpu.async_copy(src2_hbm.at[crow + ci + 8], sidxr.at[u8],
                                 si_[u8])
                pltpu.async_copy(dst2_hbm.at[crow + ci + 8], didxr.at[u8],
                                 si_[u8])

            @pl.when(ci + 3 < _CPT)
            def _issue_gather():
                pltpu.make_async_copy(src2_hbm.at[crow + ci + 3],
                                      sidxr.at[u2], si_[u2]).wait()
                pltpu.make_async_copy(dst2_hbm.at[crow + ci + 3],
                                      didxr.at[u2], si_[u2]).wait()
                pltpu.async_copy(table_hbm.at[sidxr.at[u2]], rows_[b2],
                                 sg_[b2])
                pltpu.async_copy(adst_hbm.at[didxr.at[u2]], arows_[b2],
                                 sg_[b2])
        return carry

    lax.fori_loop(0, _CPT // _IDXN, round_body, 0)
    for cj in (_CPT - 2, _CPT - 1):
        b = cj % _NB1
        pltpu.make_async_copy(rows_[b], acc.at[didxr.at[cj % _IDXN]],
                              ss_[b]).wait()
    plsc.subcore_barrier()
    pltpu.sync_copy(acc.at[pl.ds(row0, _RPT)], out_hbm.at[cid * _NS + sid])


def _make_sc_l1(D, F):
    mesh = plsc.VectorSubcoreMesh(core_axis_name="c", subcore_axis_name="s",
                                  num_cores=_NC, num_subcores=_NS)
    return pl.kernel(
        functools.partial(_sc_edge_body_l1, D=D, F=F),
        out_type=jax.ShapeDtypeStruct((_NW, _RPT, D), jnp.float32),
        mesh=mesh,
        compiler_params=pltpu.CompilerParams(use_tc_tiling_on_sc=False),
        scratch_types=(
            [pltpu.VMEM((_IDXN, _B), jnp.int32)] * 2
            + [pltpu.VMEM((_B, D), jnp.float32)] * _NB1
            + [pltpu.VMEM((_B, F), jnp.float32)] * _NB1
            + [pltpu.VMEM_SHARED((_N, D), jnp.float32)]
            + [pltpu.SemaphoreType.DMA] * (2 * _NB1 + _IDXN)
        ),
    )


def _sc_edge_body(table_hbm, adst_hbm, src2_hbm, dst2_hbm, zeros_hbm, out_hbm,
                  *scr, D, F, NBUF):
    sidx, didx = scr[0], scr[1]
    rows_ = scr[2:2 + NBUF]
    arows_ = scr[2 + NBUF:2 + 2 * NBUF]
    obuf_ = scr[2 + 2 * NBUF:2 + 3 * NBUF]
    acc = scr[2 + 3 * NBUF]
    sg_ = scr[3 + 3 * NBUF:3 + 4 * NBUF]
    ss_ = scr[3 + 4 * NBUF:3 + 5 * NBUF]
    cid = lax.axis_index("c")
    sid = lax.axis_index("s")
    wid = sid * _NC + cid
    nblk = F // 16

    row0 = pl.multiple_of(sid * _RPT, _RPT)
    pltpu.sync_copy(zeros_hbm, acc.at[pl.ds(row0, _RPT)])
    plsc.subcore_barrier()

    crow = pl.multiple_of(wid * _CPT, 2)
    pltpu.sync_copy(src2_hbm.at[pl.ds(crow, _CPT)], sidx)
    pltpu.sync_copy(dst2_hbm.at[pl.ds(crow, _CPT)], didx)

    for b in range(NBUF):
        pltpu.async_copy(table_hbm.at[sidx.at[b]], rows_[b], sg_[b])
        pltpu.async_copy(adst_hbm.at[didx.at[b]], arows_[b], sg_[b])

    def round_body(pi, carry):
        for b in range(NBUF):
            ci = pi * NBUF + b
            rws, ars, obf = rows_[b], arows_[b], obuf_[b]
            pltpu.make_async_copy(table_hbm.at[sidx.at[ci]], rws, sg_[b]).wait()
            pltpu.make_async_copy(adst_hbm.at[didx.at[ci]], ars, sg_[b]).wait()

            @pl.when(ci >= NBUF)
            def _wait_prev_scatter():
                pltpu.make_async_copy(obf, acc.at[didx.at[ci]], ss_[b]).wait()

            for e in range(_B):
                for k in range(nblk):
                    a = rws[e, pl.ds(F + k * 16, 16)] + ars[e, pl.ds(k * 16, 16)]
                    a = jnp.maximum(a, 0.2 * a)
                    wv = jnp.exp(a)
                    obf[e, pl.ds(F + k * 16, 16)] = wv
                    obf[e, pl.ds(k * 16, 16)] = rws[e, pl.ds(k * 16, 16)] * wv
            pltpu.async_copy(obf, acc.at[didx.at[ci]], ss_[b], add=True)

            @pl.when(ci + NBUF < _CPT)
            def _issue_next_gather():
                pltpu.async_copy(table_hbm.at[sidx.at[ci + NBUF]], rws, sg_[b])
                pltpu.async_copy(adst_hbm.at[didx.at[ci + NBUF]], ars, sg_[b])
        return carry

    lax.fori_loop(0, _CPT // NBUF, round_body, 0)
    for b in range(NBUF):
        pltpu.make_async_copy(obuf_[b], acc.at[didx.at[b]], ss_[b]).wait()
    plsc.subcore_barrier()
    pltpu.sync_copy(acc.at[pl.ds(row0, _RPT)], out_hbm.at[cid * _NS + sid])


def _make_sc(D, F, NBUF):
    assert _CPT % NBUF == 0
    mesh = plsc.VectorSubcoreMesh(core_axis_name="c", subcore_axis_name="s",
                                  num_cores=_NC, num_subcores=_NS)
    return pl.kernel(
        functools.partial(_sc_edge_body, D=D, F=F, NBUF=NBUF),
        out_type=jax.ShapeDtypeStruct((_NW, _RPT, D), jnp.float32),
        mesh=mesh,
        compiler_params=pltpu.CompilerParams(use_tc_tiling_on_sc=False),
        scratch_types=(
            [pltpu.VMEM((_CPT, _B), jnp.int32)] * 2
            + [pltpu.VMEM((_B, D), jnp.float32)] * NBUF
            + [pltpu.VMEM((_B, F), jnp.float32)] * NBUF
            + [pltpu.VMEM((_B, D), jnp.float32)] * NBUF
            + [pltpu.VMEM_SHARED((_N, D), jnp.float32)]
            + [pltpu.SemaphoreType.DMA] * (2 * NBUF)
        ),
    )


_sc_layer1 = _make_sc_l1(_D1, _F1)
_sc_layer2 = _make_sc(_D2, _F2, 10)


def kernel(x, edge_index, mlp_w1, mlp_b1, mlp_w2, mlp_b2, w1, att_src1,
           att_dst1, b1, w2, att_src2, att_dst2, b2):
    f32 = jnp.float32
    eye8 = jnp.eye(8, dtype=f32)
    As = (att_src1[:, :, None] * eye8[:, None, :]).reshape(_F1, _HEADS)
    Ad = (att_dst1[:, :, None] * eye8[:, None, :]).reshape(_F1, _HEADS)
    Rep = jnp.repeat(eye8, 8, axis=1)
    Asx = As @ Rep
    Adx = Ad @ Rep
    ones16 = jnp.ones((1, _F2), f32)
    As2x = att_src2.reshape(-1, 1) @ ones16
    Ad2x = att_dst2.reshape(-1, 1) @ ones16
    mb1 = mlp_b1.reshape(1, -1)
    mb2 = mlp_b2.reshape(1, -1)
    b1r = b1.reshape(1, -1)
    b2r = b2.reshape(1, -1)
    z1 = jnp.zeros((_RPT, _D1), f32)
    z2 = jnp.zeros((_RPT, _D2), f32)
    src2 = edge_index[0].reshape(_E // _B, _B)
    dst2 = edge_index[1].reshape(_E // _B, _B)

    tab1, adst1 = pl.pallas_call(
        _tc1_body,
        out_shape=[jax.ShapeDtypeStruct((_N, _D1), f32),
                   jax.ShapeDtypeStruct((_N, _F1), f32)],
    )(x, mlp_w1, mb1, mlp_w2, mb2, w1, Asx, Adx)

    p1 = _sc_layer1(tab1, adst1, src2, dst2, z1).reshape(_NC, _N, _D1)

    tab2, adst2 = pl.pallas_call(
        _tc2_body,
        out_shape=[jax.ShapeDtypeStruct((_N, _D2), f32),
                   jax.ShapeDtypeStruct((_N, _F2), f32)],
    )(p1, b1r, w2, As2x, Ad2x)

    p2 = _sc_layer2(tab2, adst2, src2, dst2, z2).reshape(_NC, _N, _D2)

    out = pl.pallas_call(
        _tc3_body,
        out_shape=jax.ShapeDtypeStruct((_N, _OUT), f32),
    )(p2, b2r)
    return out


# speedup vs baseline: 1.1477x; 1.1477x over previous
import functools

import jax
import jax.numpy as jnp
from jax import lax
from jax.experimental import pallas as pl
from jax.experimental.pallas import tpu as pltpu
from jax.experimental.pallas import tpu_sc as plsc

_N = 10000
_E = 320000
_IN = 128
_AUG = 6
_DIN = _IN - _AUG
_HEADS = 8
_HID = 8
_OUT = 16

_D1 = 128
_F1 = 64
_D2 = 32
_F2 = 16

_NC = 2
_NS = 16
_NW = _NC * _NS
_EPT = _E // _NW
_B = 40
_CPT = _EPT // _B
_PAIRS = _CPT // 2
_RPT = _N // _NS


def _tc1_body(x_ref, mw1_ref, mb1_ref, mw2_ref, mb2_ref, w1_ref, as_ref,
              ad_ref, tab_ref, adst_ref):
    x = x_ref[...]
    orig = x[:, :_DIN]
    app = x[:, _DIN:]
    mean = jnp.mean(app, axis=0, keepdims=True)
    cent = app - mean
    var = jnp.sum(cent * cent, axis=0, keepdims=True) / (_N - 1)
    z = cent / jnp.sqrt(var)
    hm = jnp.maximum(
        jnp.dot(z, mw1_ref[...], preferred_element_type=jnp.float32)
        + mb1_ref[...], 0.0)
    s = jnp.dot(hm, mw2_ref[...], preferred_element_type=jnp.float32) + mb2_ref[...]
    scale = 1.0 / (1.0 + jnp.exp(-s))
    h = orig * (1.0 + scale)
    h1 = jnp.dot(h, w1_ref[...], preferred_element_type=jnp.float32)
    a_srcx = jnp.dot(h1, as_ref[...], preferred_element_type=jnp.float32)
    a_dstx = jnp.dot(h1, ad_ref[...], preferred_element_type=jnp.float32)
    tab_ref[...] = jnp.concatenate(
        [jnp.concatenate([h1[:, :_F1h], a_srcx[:, :_F1h]], axis=1),
         jnp.concatenate([h1[:, _F1h:], a_srcx[:, _F1h:]], axis=1)], axis=0)
    adst_ref[...] = jnp.concatenate(
        [a_dstx[:, :_F1h], a_dstx[:, _F1h:]], axis=0)


def _tc2_body(p_ref, b1_ref, w2_ref, as2_ref, ad2_ref, tab2_ref, adst2_ref):
    numer = jnp.concatenate([p_ref[0][:, :_F1h], p_ref[1][:, :_F1h]], axis=1)
    dexp = jnp.concatenate([p_ref[0][:, _F1h:], p_ref[1][:, _F1h:]], axis=1)
    h1o = jnp.maximum(numer / (dexp + 1e-16) + b1_ref[...], 0.0)
    h2 = jnp.dot(h1o, w2_ref[...], preferred_element_type=jnp.float32)
    a2sx = jnp.dot(h2, as2_ref[...], preferred_element_type=jnp.float32)
    a2dx = jnp.dot(h2, ad2_ref[...], preferred_element_type=jnp.float32)
    tab2_ref[...] = jnp.concatenate([h2, a2sx], axis=1)
    adst2_ref[...] = a2dx


def _tc3_body(p_ref, b2_ref, out_ref):
    p = p_ref[0] + p_ref[1]
    numer = p[:, :_F2]
    den = p[:, _F2:_F2 + 1]
    o = numer / (den + 1e-16) + b2_ref[...]
    m = jnp.max(o, axis=1, keepdims=True)
    lse = jnp.log(jnp.sum(jnp.exp(o - m), axis=1, keepdims=True)) + m
    out_ref[...] = o - lse


_B1 = 32
_ECH1 = _E // _B1
_CPT1 = _ECH1 // _NS
_NB1 = 5
_D1h = _D1 // 2
_F1h = _F1 // 2


def _sc_edge_body_l1(tabcat_hbm, adstcat_hbm, srco_hbm, dsto_hbm, dst2_hbm,
                     zeros_hbm, out_hbm, *scr):
    sidxg, didxg, didxs = scr[0], scr[1], scr[2]
    rows_ = scr[3:3 + _NB1]
    arows_ = scr[3 + _NB1:3 + 2 * _NB1]
    obuf_ = scr[3 + 2 * _NB1:3 + 3 * _NB1]
    acc = scr[3 + 3 * _NB1]
    sg_ = scr[4 + 3 * _NB1:4 + 4 * _NB1]
    ss_ = scr[4 + 4 * _NB1:4 + 5 * _NB1]
    cid = lax.axis_index("c")
    sid = lax.axis_index("s")

    row0 = pl.multiple_of(sid * _RPT, _RPT)
    pltpu.sync_copy(zeros_hbm, acc.at[pl.ds(row0, _RPT)])
    plsc.subcore_barrier()

    crow = pl.multiple_of(sid * _CPT1, 1)
    pltpu.sync_copy(srco_hbm.at[cid, pl.ds(crow, _CPT1)], sidxg)
    pltpu.sync_copy(dsto_hbm.at[cid, pl.ds(crow, _CPT1)], didxg)
    pltpu.sync_copy(dst2_hbm.at[pl.ds(crow, _CPT1)], didxs)

    for b in range(_NB1):
        pltpu.async_copy(tabcat_hbm.at[sidxg.at[b]], rows_[b], sg_[b])
        pltpu.async_copy(adstcat_hbm.at[didxg.at[b]], arows_[b], sg_[b])

    def round_body(ri, carry):
        for b in range(_NB1):
            ci = ri * _NB1 + b
            rws, ars, obf = rows_[b], arows_[b], obuf_[b]
            pltpu.make_async_copy(tabcat_hbm.at[sidxg.at[ci]], rws,
                                  sg_[b]).wait()
            pltpu.make_async_copy(adstcat_hbm.at[didxg.at[ci]], ars,
                                  sg_[b]).wait()

            @pl.when(ci >= _NB1)
            def _wait_prev_scatter():
                pltpu.make_async_copy(obf, acc.at[didxs.at[ci]], ss_[b]).wait()

            for e in range(_B1):
                for k in range(_F1h // 16):
                    a = (rws[e, pl.ds(_F1h + k * 16, 16)]
                         + ars[e, pl.ds(k * 16, 16)])
                    a = jnp.maximum(a, 0.2 * a)
                    wv = jnp.exp(a)
                    obf[e, pl.ds(_F1h + k * 16, 16)] = wv
                    obf[e, pl.ds(k * 16, 16)] = rws[e, pl.ds(k * 16, 16)] * wv
            pltpu.async_copy(obf, acc.at[didxs.at[ci]], ss_[b], add=True)

            @pl.when(ci + _NB1 < _CPT1)
            def _issue_next_gather():
                pltpu.async_copy(tabcat_hbm.at[sidxg.at[ci + _NB1]], rws,
                                 sg_[b])
                pltpu.async_copy(adstcat_hbm.at[didxg.at[ci + _NB1]], ars,
                                 sg_[b])
        return carry

    lax.fori_loop(0, _CPT1 // _NB1, round_body, 0)
    for b in range(_NB1):
        pltpu.make_async_copy(obuf_[b], acc.at[didxs.at[b]], ss_[b]).wait()
    plsc.subcore_barrier()
    pltpu.sync_copy(acc.at[pl.ds(row0, _RPT)], out_hbm.at[cid * _NS + sid])


def _make_sc_l1():
    mesh = plsc.VectorSubcoreMesh(core_axis_name="c", subcore_axis_name="s",
                                  num_cores=_NC, num_subcores=_NS)
    return pl.kernel(
        _sc_edge_body_l1,
        out_type=jax.ShapeDtypeStruct((_NW, _RPT, _D1h), jnp.float32),
        mesh=mesh,
        compiler_params=pltpu.CompilerParams(use_tc_tiling_on_sc=False),
        scratch_types=(
            [pltpu.VMEM((_CPT1, _B1), jnp.int32)] * 3
            + [pltpu.VMEM((_B1, _D1h), jnp.float32)] * _NB1
            + [pltpu.VMEM((_B1, _F1h), jnp.float32)] * _NB1
            + [pltpu.VMEM((_B1, _D1h), jnp.float32)] * _NB1
            + [pltpu.VMEM_SHARED((_N, _D1h), jnp.float32)]
            + [pltpu.SemaphoreType.DMA] * (2 * _NB1)
        ),
    )


def _sc_edge_body(table_hbm, adst_hbm, src2_hbm, dst2_hbm, zeros_hbm, out_hbm,
                  *scr, D, F, NBUF):
    sidx, didx = scr[0], scr[1]
    rows_ = scr[2:2 + NBUF]
    arows_ = scr[2 + NBUF:2 + 2 * NBUF]
    obuf_ = scr[2 + 2 * NBUF:2 + 3 * NBUF]
    acc = scr[2 + 3 * NBUF]
    sg_ = scr[3 + 3 * NBUF:3 + 4 * NBUF]
    ss_ = scr[3 + 4 * NBUF:3 + 5 * NBUF]
    cid = lax.axis_index("c")
    sid = lax.axis_index("s")
    wid = sid * _NC + cid
    nblk = F // 16

    row0 = pl.multiple_of(sid * _RPT, _RPT)
    pltpu.sync_copy(zeros_hbm, acc.at[pl.ds(row0, _RPT)])
    plsc.subcore_barrier()

    crow = pl.multiple_of(wid * _CPT, 2)
    pltpu.sync_copy(src2_hbm.at[pl.ds(crow, _CPT)], sidx)
    pltpu.sync_copy(dst2_hbm.at[pl.ds(crow, _CPT)], didx)

    for b in range(NBUF):
        pltpu.async_copy(table_hbm.at[sidx.at[b]], rows_[b], sg_[b])
        pltpu.async_copy(adst_hbm.at[didx.at[b]], arows_[b], sg_[b])

    def round_body(pi, carry):
        for b in range(NBUF):
            ci = pi * NBUF + b
            rws, ars, obf = rows_[b], arows_[b], obuf_[b]
            pltpu.make_async_copy(table_hbm.at[sidx.at[ci]], rws, sg_[b]).wait()
            pltpu.make_async_copy(adst_hbm.at[didx.at[ci]], ars, sg_[b]).wait()

            @pl.when(ci >= NBUF)
            def _wait_prev_scatter():
                pltpu.make_async_copy(obf, acc.at[didx.at[ci]], ss_[b]).wait()

            for e in range(_B):
                for k in range(nblk):
                    a = rws[e, pl.ds(F + k * 16, 16)] + ars[e, pl.ds(k * 16, 16)]
                    a = jnp.maximum(a, 0.2 * a)
                    wv = jnp.exp(a)
                    obf[e, pl.ds(F + k * 16, 16)] = wv
                    obf[e, pl.ds(k * 16, 16)] = rws[e, pl.ds(k * 16, 16)] * wv
            pltpu.async_copy(obf, acc.at[didx.at[ci]], ss_[b], add=True)

            @pl.when(ci + NBUF < _CPT)
            def _issue_next_gather():
                pltpu.async_copy(table_hbm.at[sidx.at[ci + NBUF]], rws, sg_[b])
                pltpu.async_copy(adst_hbm.at[didx.at[ci + NBUF]], ars, sg_[b])
        return carry

    lax.fori_loop(0, _CPT // NBUF, round_body, 0)
    for b in range(NBUF):
        pltpu.make_async_copy(obuf_[b], acc.at[didx.at[b]], ss_[b]).wait()
    plsc.subcore_barrier()
    pltpu.sync_copy(acc.at[pl.ds(row0, _RPT)], out_hbm.at[cid * _NS + sid])


def _make_sc(D, F, NBUF):
    assert _CPT % NBUF == 0
    mesh = plsc.VectorSubcoreMesh(core_axis_name="c", subcore_axis_name="s",
                                  num_cores=_NC, num_subcores=_NS)
    return pl.kernel(
        functools.partial(_sc_edge_body, D=D, F=F, NBUF=NBUF),
        out_type=jax.ShapeDtypeStruct((_NW, _RPT, D), jnp.float32),
        mesh=mesh,
        compiler_params=pltpu.CompilerParams(use_tc_tiling_on_sc=False),
        scratch_types=(
            [pltpu.VMEM((_CPT, _B), jnp.int32)] * 2
            + [pltpu.VMEM((_B, D), jnp.float32)] * NBUF
            + [pltpu.VMEM((_B, F), jnp.float32)] * NBUF
            + [pltpu.VMEM((_B, D), jnp.float32)] * NBUF
            + [pltpu.VMEM_SHARED((_N, D), jnp.float32)]
            + [pltpu.SemaphoreType.DMA] * (2 * NBUF)
        ),
    )


_sc_layer1 = _make_sc_l1()
_sc_layer2 = _make_sc(_D2, _F2, 10)


def kernel(x, edge_index, mlp_w1, mlp_b1, mlp_w2, mlp_b2, w1, att_src1,
           att_dst1, b1, w2, att_src2, att_dst2, b2):
    f32 = jnp.float32
    eye8 = jnp.eye(8, dtype=f32)
    As = (att_src1[:, :, None] * eye8[:, None, :]).reshape(_F1, _HEADS)
    Ad = (att_dst1[:, :, None] * eye8[:, None, :]).reshape(_F1, _HEADS)
    Rep = jnp.repeat(eye8, 8, axis=1)
    Asx = As @ Rep
    Adx = Ad @ Rep
    ones16 = jnp.ones((1, _F2), f32)
    As2x = att_src2.reshape(-1, 1) @ ones16
    Ad2x = att_dst2.reshape(-1, 1) @ ones16
    mb1 = mlp_b1.reshape(1, -1)
    mb2 = mlp_b2.reshape(1, -1)
    b1r = b1.reshape(1, -1)
    b2r = b2.reshape(1, -1)
    z1 = jnp.zeros((_RPT, _D1h), f32)
    z2 = jnp.zeros((_RPT, _D2), f32)
    src2 = edge_index[0].reshape(_E // _B, _B)
    dst2 = edge_index[1].reshape(_E // _B, _B)
    src1 = edge_index[0].reshape(_ECH1, _B1)
    dst1 = edge_index[1].reshape(_ECH1, _B1)
    srco = jnp.stack([src1, src1 + _N])
    dsto = jnp.stack([dst1, dst1 + _N])

    tab1, adst1 = pl.pallas_call(
        _tc1_body,
        out_shape=[jax.ShapeDtypeStruct((2 * _N, _D1h), f32),
                   jax.ShapeDtypeStruct((2 * _N, _F1h), f32)],
    )(x, mlp_w1, mb1, mlp_w2, mb2, w1, Asx, Adx)

    p1 = _sc_layer1(tab1, adst1, srco, dsto, dst1, z1).reshape(_NC, _N, _D1h)

    tab2, adst2 = pl.pallas_call(
        _tc2_body,
        out_shape=[jax.ShapeDtypeStruct((_N, _D2), f32),
                   jax.ShapeDtypeStruct((_N, _F2), f32)],
    )(p1, b1r, w2, As2x, Ad2x)

    p2 = _sc_layer2(tab2, adst2, src2, dst2, z2).reshape(_NC, _N, _D2)

    out = pl.pallas_call(
        _tc3_body,
        out_shape=jax.ShapeDtypeStruct((_N, _OUT), f32),
    )(p2, b2r)
    return out


# speedup vs baseline: 1.2695x; 1.1062x over previous
import functools

import jax
import jax.numpy as jnp
from jax import lax
from jax.experimental import pallas as pl
from jax.experimental.pallas import tpu as pltpu
from jax.experimental.pallas import tpu_sc as plsc

_N = 10000
_E = 320000
_IN = 128
_AUG = 6
_DIN = _IN - _AUG
_HEADS = 8
_HID = 8
_OUT = 16

_D1 = 128
_F1 = 64
_D2 = 32
_F2 = 16

_NC = 2
_NS = 16
_NW = _NC * _NS
_EPT = _E // _NW
_B = 40
_CPT = _EPT // _B
_PAIRS = _CPT // 2
_RPT = _N // _NS


def _tc1_body(x_ref, mw1_ref, mb1_ref, mw2_ref, mb2_ref, w1_ref, as_ref,
              ad_ref, tab_ref, adst_ref):
    x = x_ref[...]
    orig = x[:, :_DIN]
    app = x[:, _DIN:]
    mean = jnp.mean(app, axis=0, keepdims=True)
    cent = app - mean
    var = jnp.sum(cent * cent, axis=0, keepdims=True) / (_N - 1)
    z = cent / jnp.sqrt(var)
    hm = jnp.maximum(
        jnp.dot(z, mw1_ref[...], preferred_element_type=jnp.float32)
        + mb1_ref[...], 0.0)
    s = jnp.dot(hm, mw2_ref[...], preferred_element_type=jnp.float32) + mb2_ref[...]
    scale = 1.0 / (1.0 + jnp.exp(-s))
    h = orig * (1.0 + scale)
    h1 = jnp.dot(h, w1_ref[...], preferred_element_type=jnp.float32)
    a_srcx = jnp.dot(h1, as_ref[...], preferred_element_type=jnp.float32)
    a_dstx = jnp.dot(h1, ad_ref[...], preferred_element_type=jnp.float32)
    tab_ref[...] = jnp.concatenate(
        [jnp.concatenate([h1[:, :_F1h], a_srcx[:, :_F1h]], axis=1),
         jnp.concatenate([h1[:, _F1h:], a_srcx[:, _F1h:]], axis=1)], axis=0)
    adst_ref[...] = jnp.concatenate(
        [a_dstx[:, :_F1h], a_dstx[:, _F1h:]], axis=0)


def _tc2_body(p_ref, b1_ref, w2_ref, as2_ref, ad2_ref, tab2_ref, adst2_ref):
    numer = jnp.concatenate([p_ref[0][:, :_F1h], p_ref[1][:, :_F1h]], axis=1)
    dexp = jnp.concatenate([p_ref[0][:, _F1h:], p_ref[1][:, _F1h:]], axis=1)
    h1o = jnp.maximum(numer / (dexp + 1e-16) + b1_ref[...], 0.0)
    h2 = jnp.dot(h1o, w2_ref[...], preferred_element_type=jnp.float32)
    a2sx = jnp.dot(h2, as2_ref[...], preferred_element_type=jnp.float32)
    a2dx = jnp.dot(h2, ad2_ref[...], preferred_element_type=jnp.float32)
    tab2_ref[...] = jnp.concatenate([h2, a2sx], axis=1)
    adst2_ref[...] = a2dx


def _tc3_body(p_ref, b2_ref, out_ref):
    p = p_ref[0] + p_ref[1]
    numer = p[:, :_F2]
    den = p[:, _F2:_F2 + 1]
    o = numer / (den + 1e-16) + b2_ref[...]
    m = jnp.max(o, axis=1, keepdims=True)
    lse = jnp.log(jnp.sum(jnp.exp(o - m), axis=1, keepdims=True)) + m
    out_ref[...] = o - lse


_B1 = 32
_ECH1 = _E // _B1
_CPT1 = _ECH1 // _NS
_NB1 = 5
_D1h = _D1 // 2
_F1h = _F1 // 2


def _sc_edge_body_l1(tabcat_hbm, adstcat_hbm, src1_hbm, dst1_hbm,
                     zeros_hbm, out_hbm, *scr):
    sidxg, didxg, didxs = scr[0], scr[1], scr[2]
    rows_ = scr[3:3 + _NB1]
    arows_ = scr[3 + _NB1:3 + 2 * _NB1]
    obuf_ = scr[3 + 2 * _NB1:3 + 3 * _NB1]
    acc = scr[3 + 3 * _NB1]
    sg_ = scr[4 + 3 * _NB1:4 + 4 * _NB1]
    ss_ = scr[4 + 4 * _NB1:4 + 5 * _NB1]
    cid = lax.axis_index("c")
    sid = lax.axis_index("s")

    row0 = pl.multiple_of(sid * _RPT, _RPT)
    pltpu.sync_copy(zeros_hbm, acc.at[pl.ds(row0, _RPT)])
    plsc.subcore_barrier()

    crow = pl.multiple_of(sid * _CPT1, 1)
    pltpu.sync_copy(src1_hbm.at[pl.ds(crow, _CPT1)], sidxg)
    pltpu.sync_copy(dst1_hbm.at[pl.ds(crow, _CPT1)], didxs)
    offv = jnp.full((16,), cid * _N, jnp.int32)

    def _adjust(r, carry):
        for j in range(_B1 // 16):
            sl = pl.ds(j * 16, 16)
            sidxg[r, sl] = sidxg[r, sl] + offv
            didxg[r, sl] = didxs[r, sl] + offv
        return carry

    lax.fori_loop(0, _CPT1, _adjust, 0)

    for b in range(_NB1):
        pltpu.async_copy(tabcat_hbm.at[sidxg.at[b]], rows_[b], sg_[b])
        pltpu.async_copy(adstcat_hbm.at[didxg.at[b]], arows_[b], sg_[b])

    def round_body(ri, carry):
        for b in range(_NB1):
            ci = ri * _NB1 + b
            rws, ars, obf = rows_[b], arows_[b], obuf_[b]
            pltpu.make_async_copy(tabcat_hbm.at[sidxg.at[ci]], rws,
                                  sg_[b]).wait()
            pltpu.make_async_copy(adstcat_hbm.at[didxg.at[ci]], ars,
                                  sg_[b]).wait()

            @pl.when(ci >= _NB1)
            def _wait_prev_scatter():
                pltpu.make_async_copy(obf, acc.at[didxs.at[ci]], ss_[b]).wait()

            for e in range(_B1):
                for k in range(_F1h // 16):
                    a = (rws[e, pl.ds(_F1h + k * 16, 16)]
                         + ars[e, pl.ds(k * 16, 16)])
                    a = jnp.maximum(a, 0.2 * a)
                    wv = jnp.exp(a)
                    obf[e, pl.ds(_F1h + k * 16, 16)] = wv
                    obf[e, pl.ds(k * 16, 16)] = rws[e, pl.ds(k * 16, 16)] * wv
            pltpu.async_copy(obf, acc.at[didxs.at[ci]], ss_[b], add=True)

            @pl.when(ci + _NB1 < _CPT1)
            def _issue_next_gather():
                pltpu.async_copy(tabcat_hbm.at[sidxg.at[ci + _NB1]], rws,
                                 sg_[b])
                pltpu.async_copy(adstcat_hbm.at[didxg.at[ci + _NB1]], ars,
                                 sg_[b])
        return carry

    lax.fori_loop(0, _CPT1 // _NB1, round_body, 0)
    for b in range(_NB1):
        pltpu.make_async_copy(obuf_[b], acc.at[didxs.at[b]], ss_[b]).wait()
    plsc.subcore_barrier()
    pltpu.sync_copy(acc.at[pl.ds(row0, _RPT)], out_hbm.at[cid * _NS + sid])


def _make_sc_l1():
    mesh = plsc.VectorSubcoreMesh(core_axis_name="c", subcore_axis_name="s",
                                  num_cores=_NC, num_subcores=_NS)
    return pl.kernel(
        _sc_edge_body_l1,
        out_type=jax.ShapeDtypeStruct((_NW, _RPT, _D1h), jnp.float32),
        mesh=mesh,
        compiler_params=pltpu.CompilerParams(use_tc_tiling_on_sc=False),
        scratch_types=(
            [pltpu.VMEM((_CPT1, _B1), jnp.int32)] * 3
            + [pltpu.VMEM((_B1, _D1h), jnp.float32)] * _NB1
            + [pltpu.VMEM((_B1, _F1h), jnp.float32)] * _NB1
            + [pltpu.VMEM((_B1, _D1h), jnp.float32)] * _NB1
            + [pltpu.VMEM_SHARED((_N, _D1h), jnp.float32)]
            + [pltpu.SemaphoreType.DMA] * (2 * _NB1)
        ),
    )


def _sc_edge_body(table_hbm, adst_hbm, src2_hbm, dst2_hbm, zeros_hbm, out_hbm,
                  *scr, D, F, NBUF):
    sidx, didx = scr[0], scr[1]
    rows_ = scr[2:2 + NBUF]
    arows_ = scr[2 + NBUF:2 + 2 * NBUF]
    obuf_ = scr[2 + 2 * NBUF:2 + 3 * NBUF]
    acc = scr[2 + 3 * NBUF]
    sg_ = scr[3 + 3 * NBUF:3 + 4 * NBUF]
    ss_ = scr[3 + 4 * NBUF:3 + 5 * NBUF]
    cid = lax.axis_index("c")
    sid = lax.axis_index("s")
    wid = sid * _NC + cid
    nblk = F // 16

    row0 = pl.multiple_of(sid * _RPT, _RPT)
    pltpu.sync_copy(zeros_hbm, acc.at[pl.ds(row0, _RPT)])
    plsc.subcore_barrier()

    crow = pl.multiple_of(wid * _CPT, 2)
    pltpu.sync_copy(src2_hbm.at[pl.ds(crow, _CPT)], sidx)
    pltpu.sync_copy(dst2_hbm.at[pl.ds(crow, _CPT)], didx)

    for b in range(NBUF):
        pltpu.async_copy(table_hbm.at[sidx.at[b]], rows_[b], sg_[b])
        pltpu.async_copy(adst_hbm.at[didx.at[b]], arows_[b], sg_[b])

    def round_body(pi, carry):
        for b in range(NBUF):
            ci = pi * NBUF + b
            rws, ars, obf = rows_[b], arows_[b], obuf_[b]
            pltpu.make_async_copy(table_hbm.at[sidx.at[ci]], rws, sg_[b]).wait()
            pltpu.make_async_copy(adst_hbm.at[didx.at[ci]], ars, sg_[b]).wait()

            @pl.when(ci >= NBUF)
            def _wait_prev_scatter():
                pltpu.make_async_copy(obf, acc.at[didx.at[ci]], ss_[b]).wait()

            for e in range(_B):
                for k in range(nblk):
                    a = rws[e, pl.ds(F + k * 16, 16)] + ars[e, pl.ds(k * 16, 16)]
                    a = jnp.maximum(a, 0.2 * a)
                    wv = jnp.exp(a)
                    obf[e, pl.ds(F + k * 16, 16)] = wv
                    obf[e, pl.ds(k * 16, 16)] = rws[e, pl.ds(k * 16, 16)] * wv
            pltpu.async_copy(obf, acc.at[didx.at[ci]], ss_[b], add=True)

            @pl.when(ci + NBUF < _CPT)
            def _issue_next_gather():
                pltpu.async_copy(table_hbm.at[sidx.at[ci + NBUF]], rws, sg_[b])
                pltpu.async_copy(adst_hbm.at[didx.at[ci + NBUF]], ars, sg_[b])
        return carry

    lax.fori_loop(0, _CPT // NBUF, round_body, 0)
    for b in range(NBUF):
        pltpu.make_async_copy(obuf_[b], acc.at[didx.at[b]], ss_[b]).wait()
    plsc.subcore_barrier()
    pltpu.sync_copy(acc.at[pl.ds(row0, _RPT)], out_hbm.at[cid * _NS + sid])


def _make_sc(D, F, NBUF):
    assert _CPT % NBUF == 0
    mesh = plsc.VectorSubcoreMesh(core_axis_name="c", subcore_axis_name="s",
                                  num_cores=_NC, num_subcores=_NS)
    return pl.kernel(
        functools.partial(_sc_edge_body, D=D, F=F, NBUF=NBUF),
        out_type=jax.ShapeDtypeStruct((_NW, _RPT, D), jnp.float32),
        mesh=mesh,
        compiler_params=pltpu.CompilerParams(use_tc_tiling_on_sc=False),
        scratch_types=(
            [pltpu.VMEM((_CPT, _B), jnp.int32)] * 2
            + [pltpu.VMEM((_B, D), jnp.float32)] * NBUF
            + [pltpu.VMEM((_B, F), jnp.float32)] * NBUF
            + [pltpu.VMEM((_B, D), jnp.float32)] * NBUF
            + [pltpu.VMEM_SHARED((_N, D), jnp.float32)]
            + [pltpu.SemaphoreType.DMA] * (2 * NBUF)
        ),
    )


_sc_layer1 = _make_sc_l1()
_sc_layer2 = _make_sc(_D2, _F2, 10)


def kernel(x, edge_index, mlp_w1, mlp_b1, mlp_w2, mlp_b2, w1, att_src1,
           att_dst1, b1, w2, att_src2, att_dst2, b2):
    f32 = jnp.float32
    eye8 = jnp.eye(8, dtype=f32)
    As = (att_src1[:, :, None] * eye8[:, None, :]).reshape(_F1, _HEADS)
    Ad = (att_dst1[:, :, None] * eye8[:, None, :]).reshape(_F1, _HEADS)
    Rep = jnp.repeat(eye8, 8, axis=1)
    Asx = As @ Rep
    Adx = Ad @ Rep
    ones16 = jnp.ones((1, _F2), f32)
    As2x = att_src2.reshape(-1, 1) @ ones16
    Ad2x = att_dst2.reshape(-1, 1) @ ones16
    mb1 = mlp_b1.reshape(1, -1)
    mb2 = mlp_b2.reshape(1, -1)
    b1r = b1.reshape(1, -1)
    b2r = b2.reshape(1, -1)
    z1 = jnp.zeros((_RPT, _D1h), f32)
    z2 = jnp.zeros((_RPT, _D2), f32)
    src2 = edge_index[0].reshape(_E // _B, _B)
    dst2 = edge_index[1].reshape(_E // _B, _B)
    src1 = edge_index[0].reshape(_ECH1, _B1)
    dst1 = edge_index[1].reshape(_ECH1, _B1)

    tab1, adst1 = pl.pallas_call(
        _tc1_body,
        out_shape=[jax.ShapeDtypeStruct((2 * _N, _D1h), f32),
                   jax.ShapeDtypeStruct((2 * _N, _F1h), f32)],
    )(x, mlp_w1, mb1, mlp_w2, mb2, w1, Asx, Adx)

    p1 = _sc_layer1(tab1, adst1, src1, dst1, z1).reshape(_NC, _N, _D1h)

    tab2, adst2 = pl.pallas_call(
        _tc2_body,
        out_shape=[jax.ShapeDtypeStruct((_N, _D2), f32),
                   jax.ShapeDtypeStruct((_N, _F2), f32)],
    )(p1, b1r, w2, As2x, Ad2x)

    p2 = _sc_layer2(tab2, adst2, src2, dst2, z2).reshape(_NC, _N, _D2)

    out = pl.pallas_call(
        _tc3_body,
        out_shape=jax.ShapeDtypeStruct((_N, _OUT), f32),
    )(p2, b2r)
    return out


# speedup vs baseline: 1.3499x; 1.0633x over previous
import functools

import jax
import jax.numpy as jnp
from jax import lax
from jax.experimental import pallas as pl
from jax.experimental.pallas import tpu as pltpu
from jax.experimental.pallas import tpu_sc as plsc

_N = 10000
_E = 320000
_IN = 128
_AUG = 6
_DIN = _IN - _AUG
_HEADS = 8
_HID = 8
_OUT = 16

_D1 = 128
_F1 = 64
_D2 = 32
_F2 = 16

_NC = 2
_NS = 16
_NW = _NC * _NS
_EPT = _E // _NW
_B2 = 80
_CPT2 = _EPT // _B2
_RPT = _N // _NS


def _tc1_body(x_ref, mw1_ref, mb1_ref, mw2_ref, mb2_ref, w1_ref, as_ref,
              ad_ref, tab_ref, adst_ref):
    x = x_ref[...]
    orig = x[:, :_DIN]
    app = x[:, _DIN:]
    mean = jnp.mean(app, axis=0, keepdims=True)
    cent = app - mean
    var = jnp.sum(cent * cent, axis=0, keepdims=True) / (_N - 1)
    z = cent / jnp.sqrt(var)
    hm = jnp.maximum(
        jnp.dot(z, mw1_ref[...], preferred_element_type=jnp.float32)
        + mb1_ref[...], 0.0)
    s = jnp.dot(hm, mw2_ref[...], preferred_element_type=jnp.float32) + mb2_ref[...]
    scale = 1.0 / (1.0 + jnp.exp(-s))
    h = orig * (1.0 + scale)
    h1 = jnp.dot(h, w1_ref[...], preferred_element_type=jnp.float32)
    a_srcx = jnp.dot(h1, as_ref[...], preferred_element_type=jnp.float32)
    a_dstx = jnp.dot(h1, ad_ref[...], preferred_element_type=jnp.float32)
    tab_ref[...] = jnp.concatenate(
        [jnp.concatenate([h1[:, :_F1h], a_srcx[:, :_F1h]], axis=1),
         jnp.concatenate([h1[:, _F1h:], a_srcx[:, _F1h:]], axis=1)], axis=0)
    adst_ref[...] = jnp.concatenate(
        [a_dstx[:, :_F1h], a_dstx[:, _F1h:]], axis=0)


def _tc2_body(p_ref, b1_ref, w2_ref, as2_ref, ad2_ref, tab2_ref, adst2_ref):
    numer = jnp.concatenate([p_ref[0][:, :_F1h], p_ref[1][:, :_F1h]], axis=1)
    dexp = jnp.concatenate([p_ref[0][:, _F1h:], p_ref[1][:, _F1h:]], axis=1)
    h1o = jnp.maximum(numer / (dexp + 1e-16) + b1_ref[...], 0.0)
    h2 = jnp.dot(h1o, w2_ref[...], preferred_element_type=jnp.float32)
    a2sx = jnp.dot(h2, as2_ref[...], preferred_element_type=jnp.float32)
    a2dx = jnp.dot(h2, ad2_ref[...], preferred_element_type=jnp.float32)
    tab2_ref[...] = jnp.concatenate([h2, a2sx], axis=1)
    adst2_ref[...] = a2dx


def _tc3_body(p_ref, b2_ref, out_ref):
    p = p_ref[0] + p_ref[1]
    numer = p[:, :_F2]
    den = p[:, _F2:_F2 + 1]
    o = numer / (den + 1e-16) + b2_ref[...]
    m = jnp.max(o, axis=1, keepdims=True)
    lse = jnp.log(jnp.sum(jnp.exp(o - m), axis=1, keepdims=True)) + m
    out_ref[...] = o - lse


_B1 = 32
_ECH1 = _E // _B1
_CPT1 = _ECH1 // _NS
_NB1 = 5
_D1h = _D1 // 2
_F1h = _F1 // 2


def _sc_edge_body_l1(tabcat_hbm, adstcat_hbm, src1_hbm, dst1_hbm,
                     zeros_hbm, out_hbm, *scr):
    sidxg, didxg, didxs = scr[0], scr[1], scr[2]
    rows_ = scr[3:3 + _NB1]
    arows_ = scr[3 + _NB1:3 + 2 * _NB1]
    obuf_ = scr[3 + 2 * _NB1:3 + 3 * _NB1]
    acc = scr[3 + 3 * _NB1]
    sg_ = scr[4 + 3 * _NB1:4 + 4 * _NB1]
    ss_ = scr[4 + 4 * _NB1:4 + 5 * _NB1]
    cid = lax.axis_index("c")
    sid = lax.axis_index("s")

    row0 = pl.multiple_of(sid * _RPT, _RPT)
    pltpu.sync_copy(zeros_hbm, acc.at[pl.ds(row0, _RPT)])
    plsc.subcore_barrier()

    crow = pl.multiple_of(sid * _CPT1, 1)
    pltpu.sync_copy(src1_hbm.at[pl.ds(crow, _CPT1)], sidxg)
    pltpu.sync_copy(dst1_hbm.at[pl.ds(crow, _CPT1)], didxs)
    offv = jnp.full((16,), cid * _N, jnp.int32)

    def _adjust(r, carry):
        for j in range(_B1 // 16):
            sl = pl.ds(j * 16, 16)
            sidxg[r, sl] = sidxg[r, sl] + offv
            didxg[r, sl] = didxs[r, sl] + offv
        return carry

    lax.fori_loop(0, _CPT1, _adjust, 0)

    for b in range(_NB1):
        pltpu.async_copy(tabcat_hbm.at[sidxg.at[b]], rows_[b], sg_[b])
        pltpu.async_copy(adstcat_hbm.at[didxg.at[b]], arows_[b], sg_[b])

    def round_body(ri, carry):
        for b in range(_NB1):
            ci = ri * _NB1 + b
            rws, ars, obf = rows_[b], arows_[b], obuf_[b]
            pltpu.make_async_copy(tabcat_hbm.at[sidxg.at[ci]], rws,
                                  sg_[b]).wait()
            pltpu.make_async_copy(adstcat_hbm.at[didxg.at[ci]], ars,
                                  sg_[b]).wait()

            @pl.when(ci >= _NB1)
            def _wait_prev_scatter():
                pltpu.make_async_copy(obf, acc.at[didxs.at[ci]], ss_[b]).wait()

            for e in range(_B1):
                for k in range(_F1h // 16):
                    a = (rws[e, pl.ds(_F1h + k * 16, 16)]
                         + ars[e, pl.ds(k * 16, 16)])
                    a = jnp.maximum(a, 0.2 * a)
                    wv = jnp.exp(a)
                    obf[e, pl.ds(_F1h + k * 16, 16)] = wv
                    obf[e, pl.ds(k * 16, 16)] = rws[e, pl.ds(k * 16, 16)] * wv
            pltpu.async_copy(obf, acc.at[didxs.at[ci]], ss_[b], add=True)

            @pl.when(ci + _NB1 < _CPT1)
            def _issue_next_gather():
                pltpu.async_copy(tabcat_hbm.at[sidxg.at[ci + _NB1]], rws,
                                 sg_[b])
                pltpu.async_copy(adstcat_hbm.at[didxg.at[ci + _NB1]], ars,
                                 sg_[b])
        return carry

    lax.fori_loop(0, _CPT1 // _NB1, round_body, 0)
    for b in range(_NB1):
        pltpu.make_async_copy(obuf_[b], acc.at[didxs.at[b]], ss_[b]).wait()
    plsc.subcore_barrier()
    pltpu.sync_copy(acc.at[pl.ds(row0, _RPT)], out_hbm.at[cid * _NS + sid])


def _make_sc_l1():
    mesh = plsc.VectorSubcoreMesh(core_axis_name="c", subcore_axis_name="s",
                                  num_cores=_NC, num_subcores=_NS)
    return pl.kernel(
        _sc_edge_body_l1,
        out_type=jax.ShapeDtypeStruct((_NW, _RPT, _D1h), jnp.float32),
        mesh=mesh,
        compiler_params=pltpu.CompilerParams(use_tc_tiling_on_sc=False),
        scratch_types=(
            [pltpu.VMEM((_CPT1, _B1), jnp.int32)] * 3
            + [pltpu.VMEM((_B1, _D1h), jnp.float32)] * _NB1
            + [pltpu.VMEM((_B1, _F1h), jnp.float32)] * _NB1
            + [pltpu.VMEM((_B1, _D1h), jnp.float32)] * _NB1
            + [pltpu.VMEM_SHARED((_N, _D1h), jnp.float32)]
            + [pltpu.SemaphoreType.DMA] * (2 * _NB1)
        ),
    )


def _sc_edge_body(table_hbm, adst_hbm, src2_hbm, dst2_hbm, zeros_hbm, out_hbm,
                  *scr, D, F, NBUF, B, CPT):
    sidx, didx = scr[0], scr[1]
    rows_ = scr[2:2 + NBUF]
    arows_ = scr[2 + NBUF:2 + 2 * NBUF]
    obuf_ = scr[2 + 2 * NBUF:2 + 3 * NBUF]
    acc = scr[2 + 3 * NBUF]
    sg_ = scr[3 + 3 * NBUF:3 + 4 * NBUF]
    ss_ = scr[3 + 4 * NBUF:3 + 5 * NBUF]
    cid = lax.axis_index("c")
    sid = lax.axis_index("s")
    wid = sid * _NC + cid
    nblk = F // 16

    row0 = pl.multiple_of(sid * _RPT, _RPT)
    pltpu.sync_copy(zeros_hbm, acc.at[pl.ds(row0, _RPT)])
    plsc.subcore_barrier()

    crow = pl.multiple_of(wid * CPT, 1)
    pltpu.sync_copy(src2_hbm.at[pl.ds(crow, CPT)], sidx)
    pltpu.sync_copy(dst2_hbm.at[pl.ds(crow, CPT)], didx)

    for b in range(NBUF):
        pltpu.async_copy(table_hbm.at[sidx.at[b]], rows_[b], sg_[b])
        pltpu.async_copy(adst_hbm.at[didx.at[b]], arows_[b], sg_[b])

    def round_body(pi, carry):
        for b in range(NBUF):
            ci = pi * NBUF + b
            rws, ars, obf = rows_[b], arows_[b], obuf_[b]
            pltpu.make_async_copy(table_hbm.at[sidx.at[ci]], rws, sg_[b]).wait()
            pltpu.make_async_copy(adst_hbm.at[didx.at[ci]], ars, sg_[b]).wait()

            @pl.when(ci >= NBUF)
            def _wait_prev_scatter():
                pltpu.make_async_copy(obf, acc.at[didx.at[ci]], ss_[b]).wait()

            for e in range(B):
                for k in range(nblk):
                    a = rws[e, pl.ds(F + k * 16, 16)] + ars[e, pl.ds(k * 16, 16)]
                    a = jnp.maximum(a, 0.2 * a)
                    wv = jnp.exp(a)
                    obf[e, pl.ds(F + k * 16, 16)] = wv
                    obf[e, pl.ds(k * 16, 16)] = rws[e, pl.ds(k * 16, 16)] * wv
            pltpu.async_copy(obf, acc.at[didx.at[ci]], ss_[b], add=True)

            @pl.when(ci + NBUF < CPT)
            def _issue_next_gather():
                pltpu.async_copy(table_hbm.at[sidx.at[ci + NBUF]], rws, sg_[b])
                pltpu.async_copy(adst_hbm.at[didx.at[ci + NBUF]], ars, sg_[b])
        return carry

    lax.fori_loop(0, CPT // NBUF, round_body, 0)
    for b in range(NBUF):
        pltpu.make_async_copy(obuf_[b], acc.at[didx.at[b]], ss_[b]).wait()
    plsc.subcore_barrier()
    pltpu.sync_copy(acc.at[pl.ds(row0, _RPT)], out_hbm.at[cid * _NS + sid])


def _make_sc(D, F, NBUF, B, CPT):
    assert CPT % NBUF == 0
    mesh = plsc.VectorSubcoreMesh(core_axis_name="c", subcore_axis_name="s",
                                  num_cores=_NC, num_subcores=_NS)
    return pl.kernel(
        functools.partial(_sc_edge_body, D=D, F=F, NBUF=NBUF, B=B, CPT=CPT),
        out_type=jax.ShapeDtypeStruct((_NW, _RPT, D), jnp.float32),
        mesh=mesh,
        compiler_params=pltpu.CompilerParams(use_tc_tiling_on_sc=False),
        scratch_types=(
            [pltpu.VMEM((CPT, B), jnp.int32)] * 2
            + [pltpu.VMEM((B, D), jnp.float32)] * NBUF
            + [pltpu.VMEM((B, F), jnp.float32)] * NBUF
            + [pltpu.VMEM((B, D), jnp.float32)] * NBUF
            + [pltpu.VMEM_SHARED((_N, D), jnp.float32)]
            + [pltpu.SemaphoreType.DMA] * (2 * NBUF)
        ),
    )


_sc_layer1 = _make_sc_l1()
_sc_layer2 = _make_sc(_D2, _F2, 5, _B2, _CPT2)


def kernel(x, edge_index, mlp_w1, mlp_b1, mlp_w2, mlp_b2, w1, att_src1,
           att_dst1, b1, w2, att_src2, att_dst2, b2):
    f32 = jnp.float32
    eye8 = jnp.eye(8, dtype=f32)
    As = (att_src1[:, :, None] * eye8[:, None, :]).reshape(_F1, _HEADS)
    Ad = (att_dst1[:, :, None] * eye8[:, None, :]).reshape(_F1, _HEADS)
    Rep = jnp.repeat(eye8, 8, axis=1)
    Asx = As @ Rep
    Adx = Ad @ Rep
    ones16 = jnp.ones((1, _F2), f32)
    As2x = att_src2.reshape(-1, 1) @ ones16
    Ad2x = att_dst2.reshape(-1, 1) @ ones16
    mb1 = mlp_b1.reshape(1, -1)
    mb2 = mlp_b2.reshape(1, -1)
    b1r = b1.reshape(1, -1)
    b2r = b2.reshape(1, -1)
    z1 = jnp.zeros((_RPT, _D1h), f32)
    z2 = jnp.zeros((_RPT, _D2), f32)
    src2 = edge_index[0].reshape(_E // _B2, _B2)
    dst2 = edge_index[1].reshape(_E // _B2, _B2)
    src1 = edge_index[0].reshape(_ECH1, _B1)
    dst1 = edge_index[1].reshape(_ECH1, _B1)

    tab1, adst1 = pl.pallas_call(
        _tc1_body,
        out_shape=[jax.ShapeDtypeStruct((2 * _N, _D1h), f32),
                   jax.ShapeDtypeStruct((2 * _N, _F1h), f32)],
    )(x, mlp_w1, mb1, mlp_w2, mb2, w1, Asx, Adx)

    p1 = _sc_layer1(tab1, adst1, src1, dst1, z1).reshape(_NC, _N, _D1h)

    tab2, adst2 = pl.pallas_call(
        _tc2_body,
        out_shape=[jax.ShapeDtypeStruct((_N, _D2), f32),
                   jax.ShapeDtypeStruct((_N, _F2), f32)],
    )(p1, b1r, w2, As2x, Ad2x)

    p2 = _sc_layer2(tab2, adst2, src2, dst2, z2).reshape(_NC, _N, _D2)

    out = pl.pallas_call(
        _tc3_body,
        out_shape=jax.ShapeDtypeStruct((_N, _OUT), f32),
    )(p2, b2r)
    return out
